# baseline (speedup 1.0000x reference)
import functools

import jax
import jax.numpy as jnp
from jax import lax
from jax.experimental import pallas as pl
from jax.experimental.pallas import tpu as pltpu
from jax.experimental.pallas import tpu_sc as plsc

_NC = 2
_NS = 16
_NW = _NC * _NS
_BN = 2048

_F32 = jnp.float32


def _pick_chunk(per_worker, cap):
    for c in (200, 128, 40, 8):
        if c <= cap and per_worker % c == 0:
            return c
    raise ValueError(f"no valid chunk for {per_worker}")


_BF16 = jnp.bfloat16


def _proj_body(x_ref, w_ref, b_ref, o_ref):
    o_ref[...] = (
        jnp.dot(x_ref[...], w_ref[0], preferred_element_type=_F32) + b_ref[0]
    )


def _proj_tc(x, wpair, bpair):
    n = x.shape[0]
    nb = n // _BN
    return pl.pallas_call(
        _proj_body,
        grid=(2 * nb,),
        in_specs=[
            pl.BlockSpec((_BN, 128), lambda i: (lax.rem(i, nb), 0)),
            pl.BlockSpec((1, 128, 128), lambda i: (i // nb, 0, 0)),
            pl.BlockSpec((1, 1, 128), lambda i: (i // nb, 0, 0)),
        ],
        out_specs=pl.BlockSpec((_BN, 128), lambda i: (i, 0)),
        out_shape=jax.ShapeDtypeStruct((2 * n, 128), _F32),
    )(x, wpair, bpair)


def _pack_cols(x):
    n = x.shape[0]
    return jax.lax.bitcast_convert_type(
        x.astype(_BF16).reshape(n, 64, 2), jnp.int32)


def _gather_sc(td, ts, src, dst):
    e = src.shape[0]
    assert e % _NW == 0
    ew = e // _NW
    chk = _pick_chunk(ew, 200)
    nchk = ew // chk
    mesh = plsc.VectorSubcoreMesh(core_axis_name="c", subcore_axis_name="s")

    def body(td_h, ts_h, src_h, dst_h, g_h,
             idx_a0, idx_a1, idx_b0, idx_b1,
             buf_a0, buf_a1, buf_b0, buf_b1, sem0, sem1):
        wid = lax.axis_index("s") * _NC + lax.axis_index("c")
        base0 = pl.multiple_of(wid * ew, 8)
        sems = (sem0, sem1)
        idx_as = (idx_a0, idx_a1)
        idx_bs = (idx_b0, idx_b1)
        buf_as = (buf_a0, buf_a1)
        buf_bs = (buf_b0, buf_b1)
        td_ref, ts_ref = td_h, ts_h

        def start(j, b):
            base = pl.multiple_of(base0 + j * chk, 8)
            pltpu.sync_copy(dst_h.at[pl.ds(base, chk)], idx_as[b])
            pltpu.sync_copy(src_h.at[pl.ds(base, chk)], idx_bs[b])
            pltpu.async_copy(td_ref.at[idx_as[b]], buf_as[b], sems[b])
            pltpu.async_copy(ts_ref.at[idx_bs[b]], buf_bs[b], sems[b])

        def finish(j, b):
            base = pl.multiple_of(base0 + j * chk, 8)
            pltpu.make_async_copy(
                td_ref.at[idx_as[b]], buf_as[b], sems[b]).wait()
            pltpu.make_async_copy(
                ts_ref.at[idx_bs[b]], buf_bs[b], sems[b]).wait()
            buf_a, buf_b = buf_as[b], buf_bs[b]

            def addrow(r, c2):
                for cc in range(8):
                    sl = pl.ds(cc * 16, 16)
                    buf_a[r, sl] = buf_a[r, sl] + buf_b[r, sl]
                return c2

            lax.fori_loop(0, chk, addrow, 0)
            pltpu.sync_copy(buf_as[b], g_h.at[pl.ds(base, chk)])

        start(0, 0)

        def chunk(j, carry):
            def stagepair(b):
                @pl.when(j + 1 < nchk)
                def _():
                    start(j + 1, 1 - b)
                finish(j, b)

            @pl.when(j % 2 == 0)
            def _():
                stagepair(0)

            @pl.when(j % 2 == 1)
            def _():
                stagepair(1)

            return carry

        lax.fori_loop(0, nchk, chunk, 0)

    call = pl.kernel(
        body,
        out_type=jax.ShapeDtypeStruct((e, 128), _F32),
        mesh=mesh,
        scratch_types=[
            pltpu.VMEM((chk,), jnp.int32),
            pltpu.VMEM((chk,), jnp.int32),
            pltpu.VMEM((chk,), jnp.int32),
            pltpu.VMEM((chk,), jnp.int32),
            pltpu.VMEM((chk, 128), _F32),
            pltpu.VMEM((chk, 128), _F32),
            pltpu.VMEM((chk, 128), _F32),
            pltpu.VMEM((chk, 128), _F32),
            pltpu.SemaphoreType.DMA,
            pltpu.SemaphoreType.DMA,
        ],
    )
    return call(td, ts, src, dst)


def _edge_body(g_ref, ea_ref, w1_ref, w2_ref, b2_ref, ga_ref, be_ref,
               eu_ref, ean_ref=True):
    ea32 = ea_ref[...].astype(_F32)
    pre = g_ref[...] + jnp.dot(ea32, w1_ref[...],
                               preferred_element_type=_F32)
    h = jnp.maximum(pre, 0.0)
    u = jnp.dot(h, w2_ref[...], preferred_element_type=_F32) + b2_ref[...]
    m = jnp.mean(u, axis=-1, keepdims=True)
    v = jnp.mean(jnp.square(u - m), axis=-1, keepdims=True)
    e2 = (u - m) / jnp.sqrt(v + 1e-5) * ga_ref[...] + be_ref[...]
    eu_ref[...] = e2
    if ean_ref is not None:
        ean_ref[...] = (ea32 + e2).astype(_BF16)


def _edge_tc(g, ea, pe, want_ea=True):
    e = g.shape[0]
    be = 2000
    assert e % be == 0
    grid = e // be
    w1c = pe['W1'][256:384]
    w2 = pe['W2']
    b2 = pe['b2'].reshape(1, 128)
    gam = pe['g'].reshape(1, 128)
    bet = pe['be'].reshape(1, 128)
    full = lambda i: (0, 0)
    row = lambda i: (i, 0)
    in_specs = [
        pl.BlockSpec((be, 128), row),
        pl.BlockSpec((be, 128), row),
        pl.BlockSpec((128, 128), full),
        pl.BlockSpec((128, 128), full),
        pl.BlockSpec((1, 128), full),
        pl.BlockSpec((1, 128), full),
        pl.BlockSpec((1, 128), full),
    ]
    if want_ea:
        body = _edge_body
        out_specs = [pl.BlockSpec((be, 128), row)] * 2
        out_shape = [jax.ShapeDtypeStruct((e, 128), _F32),
                     jax.ShapeDtypeStruct((e, 128), _BF16)]
    else:
        body = functools.partial(_edge_body, ean_ref=None)
        out_specs = pl.BlockSpec((be, 128), row)
        out_shape = jax.ShapeDtypeStruct((e, 128), _F32)
    return pl.pallas_call(
        body, grid=(grid,), in_specs=in_specs,
        out_specs=out_specs, out_shape=out_shape,
    )(g, ea, w1c, w2, b2, gam, bet)


def _scatter_sc(eu, dst, n_rows):
    e = eu.shape[0]
    assert e % _NW == 0
    ew = e // _NW
    chk = _pick_chunk(ew, 200)
    nchk = ew // chk
    rt = n_rows // _NS
    zr = 64
    assert rt % zr == 0
    mesh = plsc.VectorSubcoreMesh(core_axis_name="c", subcore_axis_name="s")

    def body(eu_h, dst_h, out_h,
             idx0, idx1, ubuf0, ubuf1, zbuf, acc, sem0, sem1):
        c = lax.axis_index("c")
        s = lax.axis_index("s")
        wid = s * _NC + c
        idxs = (idx0, idx1)
        ubufs = (ubuf0, ubuf1)
        sems = (sem0, sem1)

        def zrow(i, carry):
            for cc in range(8):
                zbuf[i, pl.ds(cc * 16, 16)] = jnp.zeros((16,), _F32)
            return carry

        lax.fori_loop(0, zr, zrow, 0)
        for q in range(rt // zr):
            pltpu.sync_copy(zbuf, acc.at[pl.ds(s * rt + q * zr, zr)])
        plsc.subcore_barrier()

        def start(j, b):
            base = pl.multiple_of(wid * ew + j * chk, 8)
            pltpu.async_copy(dst_h.at[pl.ds(base, chk)], idxs[b], sems[b])
            pltpu.async_copy(eu_h.at[pl.ds(base, chk)], ubufs[b], sems[b])

        def finish(j, b):
            base = pl.multiple_of(wid * ew + j * chk, 8)
            pltpu.make_async_copy(
                dst_h.at[pl.ds(base, chk)], idxs[b], sems[b]).wait()
            pltpu.make_async_copy(
                eu_h.at[pl.ds(base, chk)], ubufs[b], sems[b]).wait()
            pltpu.sync_copy(ubufs[b], acc.at[idxs[b]], add=True)

        start(0, 0)

        def chunk(j, carry):
            def stagepair(b):
                @pl.when(j + 1 < nchk)
                def _():
                    start(j + 1, 1 - b)
                finish(j, b)

            @pl.when(j % 2 == 0)
            def _():
                stagepair(0)

            @pl.when(j % 2 == 1)
            def _():
                stagepair(1)

            return carry

        lax.fori_loop(0, nchk, chunk, 0)
        plsc.subcore_barrier()
        pltpu.sync_copy(acc.at[pl.ds(s * rt, rt)],
                        out_h.at[c, pl.ds(s * rt, rt)])

    call = pl.kernel(
        body,
        out_type=jax.ShapeDtypeStruct((2, n_rows, 128), _F32),
        mesh=mesh,
        scratch_types=[
            pltpu.VMEM((chk,), jnp.int32),
            pltpu.VMEM((chk,), jnp.int32),
            pltpu.VMEM((chk, 128), _F32),
            pltpu.VMEM((chk, 128), _F32),
            pltpu.VMEM((zr, 128), _F32),
            pltpu.VMEM_SHARED((n_rows, 128), _F32),
            pltpu.SemaphoreType.DMA,
            pltpu.SemaphoreType.DMA,
        ],
    )
    return call(eu, dst)


def _node_body(x_ref, pp_ref, w1a_ref, w1b_ref, b1_ref, w2_ref, b2_ref,
               ga_ref, be_ref, o_ref):
    agg = pp_ref[0] + pp_ref[1]
    pre = (jnp.dot(x_ref[...], w1a_ref[...], preferred_element_type=_F32)
           + jnp.dot(agg, w1b_ref[...], preferred_element_type=_F32)
           + b1_ref[...])
    h = jnp.maximum(pre, 0.0)
    u = jnp.dot(h, w2_ref[...], preferred_element_type=_F32) + b2_ref[...]
    m = jnp.mean(u, axis=-1, keepdims=True)
    v = jnp.mean(jnp.square(u - m), axis=-1, keepdims=True)
    e2 = (u - m) / jnp.sqrt(v + 1e-5) * ga_ref[...] + be_ref[...]
    o_ref[...] = x_ref[...] + e2


def _node_tc(x, partials, pn):
    n = x.shape[0]
    grid = n // _BN
    full = lambda i: (0, 0)
    row = lambda i: (i, 0)
    return pl.pallas_call(
        _node_body,
        grid=(grid,),
        in_specs=[
            pl.BlockSpec((_BN, 128), row),
            pl.BlockSpec((2, _BN, 128), lambda i: (0, i, 0)),
            pl.BlockSpec((128, 128), full),
            pl.BlockSpec((128, 128), full),
            pl.BlockSpec((1, 128), full),
            pl.BlockSpec((128, 128), full),
            pl.BlockSpec((1, 128), full),
            pl.BlockSpec((1, 128), full),
            pl.BlockSpec((1, 128), full),
        ],
        out_specs=pl.BlockSpec((_BN, 128), row),
        out_shape=jax.ShapeDtypeStruct((n, 128), _F32),
    )(x, partials, pn['W1'][:128], pn['W1'][128:],
      pn['b1'].reshape(1, 128), pn['W2'], pn['b2'].reshape(1, 128),
      pn['g'].reshape(1, 128), pn['be'].reshape(1, 128))


def _pad_rows(x, n):
    return jnp.zeros((n, 128), _F32).at[:x.shape[0]].set(x)


def kernel(x_mesh, x_object, edge_index_mo, edge_index_om,
           edge_attr_mo, edge_attr_om, params):
    nm0, no0 = x_mesh.shape[0], x_object.shape[0]
    nm = -(-nm0 // _BN) * _BN
    no = -(-no0 // _BN) * _BN
    xm = _pad_rows(x_mesh, nm)
    xo = _pad_rows(x_object, no)
    smo, dmo = edge_index_mo[0], edge_index_mo[1]
    som, dom = edge_index_om[0], edge_index_om[1]
    zb = jnp.zeros((128,), _F32)

    def step(carry, p, last):
        xm, xo, ea_mo, ea_om = carry
        e_mo, e_om = p['mo']['edge'], p['om']['edge']
        t_mesh = _proj_tc(
            xm,
            jnp.stack([e_mo['W1'][128:256], e_om['W1'][0:128]]),
            jnp.stack([zb, e_om['b1']])[:, None, :])
        t_obj = _proj_tc(
            xo,
            jnp.stack([e_mo['W1'][0:128], e_om['W1'][128:256]]),
            jnp.stack([e_mo['b1'], zb])[:, None, :])
        tsmo, tdom = t_mesh[:nm], t_mesh[nm:]
        tdmo, tsom = t_obj[:no], t_obj[no:]
        g_mo = _gather_sc(tdmo, tsmo, smo, dmo)
        g_om = _gather_sc(tdom, tsom, som, dom)
        if last:
            eu_mo = _edge_tc(g_mo, ea_mo, e_mo, want_ea=False)
            eu_om = _edge_tc(g_om, ea_om, e_om, want_ea=False)
        else:
            eu_mo, ea_mo = _edge_tc(g_mo, ea_mo, e_mo)
            eu_om, ea_om = _edge_tc(g_om, ea_om, e_om)
        p_obj = _scatter_sc(eu_mo, dmo, no)
        p_mesh = _scatter_sc(eu_om, dom, nm)
        xo2 = _node_tc(xo, p_obj, p['mo']['node'])
        xm2 = _node_tc(xm, p_mesh, p['om']['node'])
        return (xm2, xo2, ea_mo, ea_om)

    carry = (xm, xo, edge_attr_mo.astype(_BF16), edge_attr_om.astype(_BF16))
    for si, p in enumerate(params):
        carry = step(carry, p, si == len(params) - 1)
    xm, xo = carry[0], carry[1]
    return jnp.concatenate([xm[:nm0], xo[:no0]], axis=0)


# speedup vs baseline: 1.0764x; 1.0764x over previous
import functools

import jax
import jax.numpy as jnp
from jax import lax
from jax.experimental import pallas as pl
from jax.experimental.pallas import tpu as pltpu
from jax.experimental.pallas import tpu_sc as plsc

_NC = 2
_NS = 16
_NW = _NC * _NS
_BN = 2048

_F32 = jnp.float32


def _pick_chunk(per_worker, cap):
    for c in (200, 128, 40, 8):
        if c <= cap and per_worker % c == 0:
            return c
    raise ValueError(f"no valid chunk for {per_worker}")


_BF16 = jnp.bfloat16


def _proj_body(x_ref, w_ref, b_ref, o_ref):
    o_ref[...] = (
        jnp.dot(x_ref[...], w_ref[0], preferred_element_type=_F32) + b_ref[0]
    )


def _proj_tc(x, wpair, bpair):
    n = x.shape[0]
    nb = n // _BN
    return pl.pallas_call(
        _proj_body,
        grid=(2 * nb,),
        in_specs=[
            pl.BlockSpec((_BN, 128), lambda i: (lax.rem(i, nb), 0)),
            pl.BlockSpec((1, 128, 128), lambda i: (i // nb, 0, 0)),
            pl.BlockSpec((1, 1, 128), lambda i: (i // nb, 0, 0)),
        ],
        out_specs=pl.BlockSpec((_BN, 128), lambda i: (i, 0)),
        out_shape=jax.ShapeDtypeStruct((2 * n, 128), _F32),
    )(x, wpair, bpair)


def _pack_cols(x):
    n = x.shape[0]
    return jax.lax.bitcast_convert_type(
        x.astype(_BF16).reshape(n, 64, 2), jnp.int32)


def _gather_sc(td, ts, src, dst):
    e = src.shape[0]
    assert e % _NW == 0
    ew = e // _NW
    chk = _pick_chunk(ew, 200)
    nchk = ew // chk
    mesh = plsc.VectorSubcoreMesh(core_axis_name="c", subcore_axis_name="s")

    def body(td_h, ts_h, src_h, dst_h, g_h,
             idx_a0, idx_a1, idx_b0, idx_b1,
             buf_a0, buf_a1, buf_b0, buf_b1, sem0, sem1):
        wid = lax.axis_index("s") * _NC + lax.axis_index("c")
        base0 = pl.multiple_of(wid * ew, 8)
        sems = (sem0, sem1)
        idx_as = (idx_a0, idx_a1)
        idx_bs = (idx_b0, idx_b1)
        buf_as = (buf_a0, buf_a1)
        buf_bs = (buf_b0, buf_b1)
        td_ref, ts_ref = td_h, ts_h

        def start(j, b):
            base = pl.multiple_of(base0 + j * chk, 8)
            pltpu.sync_copy(dst_h.at[pl.ds(base, chk)], idx_as[b])
            pltpu.sync_copy(src_h.at[pl.ds(base, chk)], idx_bs[b])
            pltpu.async_copy(td_ref.at[idx_as[b]], buf_as[b], sems[b])
            pltpu.async_copy(ts_ref.at[idx_bs[b]], buf_bs[b], sems[b])

        def finish(j, b):
            base = pl.multiple_of(base0 + j * chk, 8)
            pltpu.make_async_copy(
                td_ref.at[idx_as[b]], buf_as[b], sems[b]).wait()
            pltpu.make_async_copy(
                ts_ref.at[idx_bs[b]], buf_bs[b], sems[b]).wait()
            buf_a, buf_b = buf_as[b], buf_bs[b]

            def addrow(r, c2):
                for cc in range(8):
                    sl = pl.ds(cc * 16, 16)
                    buf_a[r, sl] = buf_a[r, sl] + buf_b[r, sl]
                return c2

            lax.fori_loop(0, chk, addrow, 0)
            pltpu.sync_copy(buf_as[b], g_h.at[pl.ds(base, chk)])

        start(0, 0)

        def chunk(j, carry):
            def stagepair(b):
                @pl.when(j + 1 < nchk)
                def _():
                    start(j + 1, 1 - b)
                finish(j, b)

            @pl.when(j % 2 == 0)
            def _():
                stagepair(0)

            @pl.when(j % 2 == 1)
            def _():
                stagepair(1)

            return carry

        lax.fori_loop(0, nchk, chunk, 0)

    call = pl.kernel(
        body,
        out_type=jax.ShapeDtypeStruct((e, 128), _F32),
        mesh=mesh,
        scratch_types=[
            pltpu.VMEM((chk,), jnp.int32),
            pltpu.VMEM((chk,), jnp.int32),
            pltpu.VMEM((chk,), jnp.int32),
            pltpu.VMEM((chk,), jnp.int32),
            pltpu.VMEM((chk, 128), _F32),
            pltpu.VMEM((chk, 128), _F32),
            pltpu.VMEM((chk, 128), _F32),
            pltpu.VMEM((chk, 128), _F32),
            pltpu.SemaphoreType.DMA,
            pltpu.SemaphoreType.DMA,
        ],
    )
    return call(td, ts, src, dst)


def _edge_body(g_ref, ea_ref, w1_ref, w2_ref, b2_ref, ga_ref, be_ref,
               eu_ref, ean_ref=True):
    ea32 = ea_ref[...].astype(_F32)
    pre = g_ref[...] + jnp.dot(ea32, w1_ref[...],
                               preferred_element_type=_F32)
    h = jnp.maximum(pre, 0.0)
    u = jnp.dot(h, w2_ref[...], preferred_element_type=_F32) + b2_ref[...]
    m = jnp.mean(u, axis=-1, keepdims=True)
    v = jnp.mean(jnp.square(u - m), axis=-1, keepdims=True)
    e2 = (u - m) / jnp.sqrt(v + 1e-5) * ga_ref[...] + be_ref[...]
    eu_ref[...] = e2
    if ean_ref is not None:
        ean_ref[...] = (ea32 + e2).astype(_BF16)


def _edge_tc(g, ea, pe, want_ea=True):
    e = g.shape[0]
    be = 4000
    assert e % be == 0
    grid = e // be
    w1c = pe['W1'][256:384]
    w2 = pe['W2']
    b2 = pe['b2'].reshape(1, 128)
    gam = pe['g'].reshape(1, 128)
    bet = pe['be'].reshape(1, 128)
    full = lambda i: (0, 0)
    row = lambda i: (i, 0)
    in_specs = [
        pl.BlockSpec((be, 128), row),
        pl.BlockSpec((be, 128), row),
        pl.BlockSpec((128, 128), full),
        pl.BlockSpec((128, 128), full),
        pl.BlockSpec((1, 128), full),
        pl.BlockSpec((1, 128), full),
        pl.BlockSpec((1, 128), full),
    ]
    if want_ea:
        body = _edge_body
        out_specs = [pl.BlockSpec((be, 128), row)] * 2
        out_shape = [jax.ShapeDtypeStruct((e, 128), _F32),
                     jax.ShapeDtypeStruct((e, 128), _BF16)]
    else:
        body = functools.partial(_edge_body, ean_ref=None)
        out_specs = pl.BlockSpec((be, 128), row)
        out_shape = jax.ShapeDtypeStruct((e, 128), _F32)
    return pl.pallas_call(
        body, grid=(grid,), in_specs=in_specs,
        out_specs=out_specs, out_shape=out_shape,
    )(g, ea, w1c, w2, b2, gam, bet)


def _scatter_sc(eu, dst, n_rows):
    e = eu.shape[0]
    assert e % _NW == 0
    ew = e // _NW
    chk = _pick_chunk(ew, 200)
    nchk = ew // chk
    rt = n_rows // _NS
    zr = 64
    assert rt % zr == 0
    mesh = plsc.VectorSubcoreMesh(core_axis_name="c", subcore_axis_name="s")

    def body(eu_h, dst_h, out_h,
             idx0, idx1, ubuf0, ubuf1, zbuf, acc, sem0, sem1):
        c = lax.axis_index("c")
        s = lax.axis_index("s")
        wid = s * _NC + c
        idxs = (idx0, idx1)
        ubufs = (ubuf0, ubuf1)
        sems = (sem0, sem1)

        def zrow(i, carry):
            for cc in range(8):
                zbuf[i, pl.ds(cc * 16, 16)] = jnp.zeros((16,), _F32)
            return carry

        lax.fori_loop(0, zr, zrow, 0)
        for q in range(rt // zr):
            pltpu.sync_copy(zbuf, acc.at[pl.ds(s * rt + q * zr, zr)])
        plsc.subcore_barrier()

        def start(j, b):
            base = pl.multiple_of(wid * ew + j * chk, 8)
            pltpu.async_copy(dst_h.at[pl.ds(base, chk)], idxs[b], sems[b])
            pltpu.async_copy(eu_h.at[pl.ds(base, chk)], ubufs[b], sems[b])

        def finish(j, b):
            base = pl.multiple_of(wid * ew + j * chk, 8)
            pltpu.make_async_copy(
                dst_h.at[pl.ds(base, chk)], idxs[b], sems[b]).wait()
            pltpu.make_async_copy(
                eu_h.at[pl.ds(base, chk)], ubufs[b], sems[b]).wait()
            pltpu.sync_copy(ubufs[b], acc.at[idxs[b]], add=True)

        start(0, 0)

        def chunk(j, carry):
            def stagepair(b):
                @pl.when(j + 1 < nchk)
                def _():
                    start(j + 1, 1 - b)
                finish(j, b)

            @pl.when(j % 2 == 0)
            def _():
                stagepair(0)

            @pl.when(j % 2 == 1)
            def _():
                stagepair(1)

            return carry

        lax.fori_loop(0, nchk, chunk, 0)
        plsc.subcore_barrier()
        pltpu.sync_copy(acc.at[pl.ds(s * rt, rt)],
                        out_h.at[c, pl.ds(s * rt, rt)])

    call = pl.kernel(
        body,
        out_type=jax.ShapeDtypeStruct((2, n_rows, 128), _F32),
        mesh=mesh,
        scratch_types=[
            pltpu.VMEM((chk,), jnp.int32),
            pltpu.VMEM((chk,), jnp.int32),
            pltpu.VMEM((chk, 128), _F32),
            pltpu.VMEM((chk, 128), _F32),
            pltpu.VMEM((zr, 128), _F32),
            pltpu.VMEM_SHARED((n_rows, 128), _F32),
            pltpu.SemaphoreType.DMA,
            pltpu.SemaphoreType.DMA,
        ],
    )
    return call(eu, dst)


def _node_body(x_ref, pp_ref, w1a_ref, w1b_ref, b1_ref, w2_ref, b2_ref,
               ga_ref, be_ref, o_ref):
    agg = pp_ref[0] + pp_ref[1]
    pre = (jnp.dot(x_ref[...], w1a_ref[...], preferred_element_type=_F32)
           + jnp.dot(agg, w1b_ref[...], preferred_element_type=_F32)
           + b1_ref[...])
    h = jnp.maximum(pre, 0.0)
    u = jnp.dot(h, w2_ref[...], preferred_element_type=_F32) + b2_ref[...]
    m = jnp.mean(u, axis=-1, keepdims=True)
    v = jnp.mean(jnp.square(u - m), axis=-1, keepdims=True)
    e2 = (u - m) / jnp.sqrt(v + 1e-5) * ga_ref[...] + be_ref[...]
    o_ref[...] = x_ref[...] + e2


def _node_tc(x, partials, pn):
    n = x.shape[0]
    grid = n // _BN
    full = lambda i: (0, 0)
    row = lambda i: (i, 0)
    return pl.pallas_call(
        _node_body,
        grid=(grid,),
        in_specs=[
            pl.BlockSpec((_BN, 128), row),
            pl.BlockSpec((2, _BN, 128), lambda i: (0, i, 0)),
            pl.BlockSpec((128, 128), full),
            pl.BlockSpec((128, 128), full),
            pl.BlockSpec((1, 128), full),
            pl.BlockSpec((128, 128), full),
            pl.BlockSpec((1, 128), full),
            pl.BlockSpec((1, 128), full),
            pl.BlockSpec((1, 128), full),
        ],
        out_specs=pl.BlockSpec((_BN, 128), row),
        out_shape=jax.ShapeDtypeStruct((n, 128), _F32),
    )(x, partials, pn['W1'][:128], pn['W1'][128:],
      pn['b1'].reshape(1, 128), pn['W2'], pn['b2'].reshape(1, 128),
      pn['g'].reshape(1, 128), pn['be'].reshape(1, 128))


def _pad_rows(x, n):
    return jnp.zeros((n, 128), _F32).at[:x.shape[0]].set(x)


def kernel(x_mesh, x_object, edge_index_mo, edge_index_om,
           edge_attr_mo, edge_attr_om, params):
    nm0, no0 = x_mesh.shape[0], x_object.shape[0]
    nm = -(-nm0 // _BN) * _BN
    no = -(-no0 // _BN) * _BN
    xm = _pad_rows(x_mesh, nm)
    xo = _pad_rows(x_object, no)
    smo, dmo = edge_index_mo[0], edge_index_mo[1]
    som, dom = edge_index_om[0], edge_index_om[1]
    zb = jnp.zeros((128,), _F32)

    def step(carry, p, last):
        xm, xo, ea_mo, ea_om = carry
        e_mo, e_om = p['mo']['edge'], p['om']['edge']
        t_mesh = _proj_tc(
            xm,
            jnp.stack([e_mo['W1'][128:256], e_om['W1'][0:128]]),
            jnp.stack([zb, e_om['b1']])[:, None, :])
        t_obj = _proj_tc(
            xo,
            jnp.stack([e_mo['W1'][0:128], e_om['W1'][128:256]]),
            jnp.stack([e_mo['b1'], zb])[:, None, :])
        tsmo, tdom = t_mesh[:nm], t_mesh[nm:]
        tdmo, tsom = t_obj[:no], t_obj[no:]
        g_mo = _gather_sc(tdmo, tsmo, smo, dmo)
        g_om = _gather_sc(tdom, tsom, som, dom)
        if last:
            eu_mo = _edge_tc(g_mo, ea_mo, e_mo, want_ea=False)
            eu_om = _edge_tc(g_om, ea_om, e_om, want_ea=False)
        else:
            eu_mo, ea_mo = _edge_tc(g_mo, ea_mo, e_mo)
            eu_om, ea_om = _edge_tc(g_om, ea_om, e_om)
        p_obj = _scatter_sc(eu_mo, dmo, no)
        p_mesh = _scatter_sc(eu_om, dom, nm)
        xo2 = _node_tc(xo, p_obj, p['mo']['node'])
        xm2 = _node_tc(xm, p_mesh, p['om']['node'])
        return (xm2, xo2, ea_mo, ea_om)

    carry = (xm, xo, edge_attr_mo.astype(_BF16), edge_attr_om.astype(_BF16))
    for si, p in enumerate(params):
        carry = step(carry, p, si == len(params) - 1)
    xm, xo = carry[0], carry[1]
    return jnp.concatenate([xm[:nm0], xo[:no0]], axis=0)


# speedup vs baseline: 1.0911x; 1.0137x over previous
import functools

import jax
import jax.numpy as jnp
from jax import lax
from jax.experimental import pallas as pl
from jax.experimental.pallas import tpu as pltpu
from jax.experimental.pallas import tpu_sc as plsc

_NC = 2
_NS = 16
_NW = _NC * _NS
_BN = 2048

_F32 = jnp.float32


def _pick_chunk(per_worker, cap):
    for c in (200, 128, 40, 8):
        if c <= cap and per_worker % c == 0:
            return c
    raise ValueError(f"no valid chunk for {per_worker}")


_BF16 = jnp.bfloat16


def _proj_body(x_ref, w_ref, b_ref, o_ref):
    o_ref[...] = (
        jnp.dot(x_ref[...], w_ref[0], preferred_element_type=_F32) + b_ref[0]
    )


def _proj_tc(x, wpair, bpair):
    n = x.shape[0]
    nb = n // _BN
    return pl.pallas_call(
        _proj_body,
        grid=(2 * nb,),
        in_specs=[
            pl.BlockSpec((_BN, 128), lambda i: (lax.rem(i, nb), 0)),
            pl.BlockSpec((1, 128, 128), lambda i: (i // nb, 0, 0)),
            pl.BlockSpec((1, 1, 128), lambda i: (i // nb, 0, 0)),
        ],
        out_specs=pl.BlockSpec((_BN, 128), lambda i: (i, 0)),
        out_shape=jax.ShapeDtypeStruct((2 * n, 128), _F32),
    )(x, wpair, bpair)


def _pack_cols(x):
    n = x.shape[0]
    return jax.lax.bitcast_convert_type(
        x.astype(_BF16).reshape(n, 64, 2), jnp.int32)


def _gather_sc(td, ts, src, dst):
    e = src.shape[0]
    assert e % _NW == 0
    ew = e // _NW
    chk = _pick_chunk(ew, 200)
    nchk = ew // chk
    mesh = plsc.VectorSubcoreMesh(core_axis_name="c", subcore_axis_name="s")

    def body(td_h, ts_h, src_h, dst_h, g_h,
             idx_a0, idx_a1, idx_b0, idx_b1,
             buf_a0, buf_a1, buf_b0, buf_b1, sem0, sem1):
        wid = lax.axis_index("s") * _NC + lax.axis_index("c")
        base0 = pl.multiple_of(wid * ew, 8)
        sems = (sem0, sem1)
        idx_as = (idx_a0, idx_a1)
        idx_bs = (idx_b0, idx_b1)
        buf_as = (buf_a0, buf_a1)
        buf_bs = (buf_b0, buf_b1)
        td_ref, ts_ref = td_h, ts_h

        def start(j, b):
            base = pl.multiple_of(base0 + j * chk, 8)
            pltpu.sync_copy(dst_h.at[pl.ds(base, chk)], idx_as[b])
            pltpu.sync_copy(src_h.at[pl.ds(base, chk)], idx_bs[b])
            pltpu.async_copy(td_ref.at[idx_as[b]], buf_as[b], sems[b])
            pltpu.async_copy(ts_ref.at[idx_bs[b]], buf_bs[b], sems[b])

        def finish(j, b):
            base = pl.multiple_of(base0 + j * chk, 8)
            pltpu.make_async_copy(
                td_ref.at[idx_as[b]], buf_as[b], sems[b]).wait()
            pltpu.make_async_copy(
                ts_ref.at[idx_bs[b]], buf_bs[b], sems[b]).wait()
            buf_a, buf_b = buf_as[b], buf_bs[b]

            def addrow(r, c2):
                for cc in range(8):
                    sl = pl.ds(cc * 16, 16)
                    buf_a[r, sl] = buf_a[r, sl] + buf_b[r, sl]
                return c2

            lax.fori_loop(0, chk, addrow, 0)
            pltpu.sync_copy(buf_as[b], g_h.at[pl.ds(base, chk)])

        start(0, 0)

        def chunk(j, carry):
            def stagepair(b):
                @pl.when(j + 1 < nchk)
                def _():
                    start(j + 1, 1 - b)
                finish(j, b)

            @pl.when(j % 2 == 0)
            def _():
                stagepair(0)

            @pl.when(j % 2 == 1)
            def _():
                stagepair(1)

            return carry

        lax.fori_loop(0, nchk, chunk, 0)

    call = pl.kernel(
        body,
        out_type=jax.ShapeDtypeStruct((e, 128), _F32),
        mesh=mesh,
        scratch_types=[
            pltpu.VMEM((chk,), jnp.int32),
            pltpu.VMEM((chk,), jnp.int32),
            pltpu.VMEM((chk,), jnp.int32),
            pltpu.VMEM((chk,), jnp.int32),
            pltpu.VMEM((chk, 128), _F32),
            pltpu.VMEM((chk, 128), _F32),
            pltpu.VMEM((chk, 128), _F32),
            pltpu.VMEM((chk, 128), _F32),
            pltpu.SemaphoreType.DMA,
            pltpu.SemaphoreType.DMA,
        ],
    )
    return call(td, ts, src, dst)


def _edge_body(g_ref, ea_ref, w1_ref, w2_ref, b2_ref, ga_ref, be_ref,
               eu_ref, ean_ref=True):
    ea32 = ea_ref[...].astype(_F32)
    pre = g_ref[...] + jnp.dot(ea32, w1_ref[...],
                               preferred_element_type=_F32)
    h = jnp.maximum(pre, 0.0)
    u = jnp.dot(h, w2_ref[...], preferred_element_type=_F32) + b2_ref[...]
    m = jnp.mean(u, axis=-1, keepdims=True)
    v = jnp.mean(jnp.square(u - m), axis=-1, keepdims=True)
    e2 = (u - m) / jnp.sqrt(v + 1e-5) * ga_ref[...] + be_ref[...]
    eu_ref[...] = e2
    if ean_ref is not None:
        ean_ref[...] = (ea32 + e2).astype(_BF16)


def _edge_tc(g, ea, pe, want_ea=True):
    e = g.shape[0]
    be = 8000
    assert e % be == 0
    grid = e // be
    w1c = pe['W1'][256:384]
    w2 = pe['W2']
    b2 = pe['b2'].reshape(1, 128)
    gam = pe['g'].reshape(1, 128)
    bet = pe['be'].reshape(1, 128)
    full = lambda i: (0, 0)
    row = lambda i: (i, 0)
    in_specs = [
        pl.BlockSpec((be, 128), row),
        pl.BlockSpec((be, 128), row),
        pl.BlockSpec((128, 128), full),
        pl.BlockSpec((128, 128), full),
        pl.BlockSpec((1, 128), full),
        pl.BlockSpec((1, 128), full),
        pl.BlockSpec((1, 128), full),
    ]
    if want_ea:
        body = _edge_body
        out_specs = [pl.BlockSpec((be, 128), row)] * 2
        out_shape = [jax.ShapeDtypeStruct((e, 128), _F32),
                     jax.ShapeDtypeStruct((e, 128), _BF16)]
    else:
        body = functools.partial(_edge_body, ean_ref=None)
        out_specs = pl.BlockSpec((be, 128), row)
        out_shape = jax.ShapeDtypeStruct((e, 128), _F32)
    return pl.pallas_call(
        body, grid=(grid,), in_specs=in_specs,
        out_specs=out_specs, out_shape=out_shape,
    )(g, ea, w1c, w2, b2, gam, bet)


def _scatter_sc(eu, dst, n_rows):
    e = eu.shape[0]
    assert e % _NW == 0
    ew = e // _NW
    chk = _pick_chunk(ew, 200)
    nchk = ew // chk
    rt = n_rows // _NS
    zr = 64
    assert rt % zr == 0
    mesh = plsc.VectorSubcoreMesh(core_axis_name="c", subcore_axis_name="s")

    def body(eu_h, dst_h, out_h,
             idx0, idx1, ubuf0, ubuf1, zbuf, acc, sem0, sem1):
        c = lax.axis_index("c")
        s = lax.axis_index("s")
        wid = s * _NC + c
        idxs = (idx0, idx1)
        ubufs = (ubuf0, ubuf1)
        sems = (sem0, sem1)

        def zrow(i, carry):
            for cc in range(8):
                zbuf[i, pl.ds(cc * 16, 16)] = jnp.zeros((16,), _F32)
            return carry

        lax.fori_loop(0, zr, zrow, 0)
        for q in range(rt // zr):
            pltpu.sync_copy(zbuf, acc.at[pl.ds(s * rt + q * zr, zr)])
        plsc.subcore_barrier()

        def start(j, b):
            base = pl.multiple_of(wid * ew + j * chk, 8)
            pltpu.async_copy(dst_h.at[pl.ds(base, chk)], idxs[b], sems[b])
            pltpu.async_copy(eu_h.at[pl.ds(base, chk)], ubufs[b], sems[b])

        def finish(j, b):
            base = pl.multiple_of(wid * ew + j * chk, 8)
            pltpu.make_async_copy(
                dst_h.at[pl.ds(base, chk)], idxs[b], sems[b]).wait()
            pltpu.make_async_copy(
                eu_h.at[pl.ds(base, chk)], ubufs[b], sems[b]).wait()
            pltpu.sync_copy(ubufs[b], acc.at[idxs[b]], add=True)

        start(0, 0)

        def chunk(j, carry):
            def stagepair(b):
                @pl.when(j + 1 < nchk)
                def _():
                    start(j + 1, 1 - b)
                finish(j, b)

            @pl.when(j % 2 == 0)
            def _():
                stagepair(0)

            @pl.when(j % 2 == 1)
            def _():
                stagepair(1)

            return carry

        lax.fori_loop(0, nchk, chunk, 0)
        plsc.subcore_barrier()
        pltpu.sync_copy(acc.at[pl.ds(s * rt, rt)],
                        out_h.at[c, pl.ds(s * rt, rt)])

    call = pl.kernel(
        body,
        out_type=jax.ShapeDtypeStruct((2, n_rows, 128), _F32),
        mesh=mesh,
        scratch_types=[
            pltpu.VMEM((chk,), jnp.int32),
            pltpu.VMEM((chk,), jnp.int32),
            pltpu.VMEM((chk, 128), _F32),
            pltpu.VMEM((chk, 128), _F32),
            pltpu.VMEM((zr, 128), _F32),
            pltpu.VMEM_SHARED((n_rows, 128), _F32),
            pltpu.SemaphoreType.DMA,
            pltpu.SemaphoreType.DMA,
        ],
    )
    return call(eu, dst)


def _node_body(x_ref, pp_ref, w1a_ref, w1b_ref, b1_ref, w2_ref, b2_ref,
               ga_ref, be_ref, o_ref):
    agg = pp_ref[0] + pp_ref[1]
    pre = (jnp.dot(x_ref[...], w1a_ref[...], preferred_element_type=_F32)
           + jnp.dot(agg, w1b_ref[...], preferred_element_type=_F32)
           + b1_ref[...])
    h = jnp.maximum(pre, 0.0)
    u = jnp.dot(h, w2_ref[...], preferred_element_type=_F32) + b2_ref[...]
    m = jnp.mean(u, axis=-1, keepdims=True)
    v = jnp.mean(jnp.square(u - m), axis=-1, keepdims=True)
    e2 = (u - m) / jnp.sqrt(v + 1e-5) * ga_ref[...] + be_ref[...]
    o_ref[...] = x_ref[...] + e2


def _node_tc(x, partials, pn):
    n = x.shape[0]
    grid = n // _BN
    full = lambda i: (0, 0)
    row = lambda i: (i, 0)
    return pl.pallas_call(
        _node_body,
        grid=(grid,),
        in_specs=[
            pl.BlockSpec((_BN, 128), row),
            pl.BlockSpec((2, _BN, 128), lambda i: (0, i, 0)),
            pl.BlockSpec((128, 128), full),
            pl.BlockSpec((128, 128), full),
            pl.BlockSpec((1, 128), full),
            pl.BlockSpec((128, 128), full),
            pl.BlockSpec((1, 128), full),
            pl.BlockSpec((1, 128), full),
            pl.BlockSpec((1, 128), full),
        ],
        out_specs=pl.BlockSpec((_BN, 128), row),
        out_shape=jax.ShapeDtypeStruct((n, 128), _F32),
    )(x, partials, pn['W1'][:128], pn['W1'][128:],
      pn['b1'].reshape(1, 128), pn['W2'], pn['b2'].reshape(1, 128),
      pn['g'].reshape(1, 128), pn['be'].reshape(1, 128))


def _pad_rows(x, n):
    return jnp.zeros((n, 128), _F32).at[:x.shape[0]].set(x)


def kernel(x_mesh, x_object, edge_index_mo, edge_index_om,
           edge_attr_mo, edge_attr_om, params):
    nm0, no0 = x_mesh.shape[0], x_object.shape[0]
    nm = -(-nm0 // _BN) * _BN
    no = -(-no0 // _BN) * _BN
    xm = _pad_rows(x_mesh, nm)
    xo = _pad_rows(x_object, no)
    smo, dmo = edge_index_mo[0], edge_index_mo[1]
    som, dom = edge_index_om[0], edge_index_om[1]
    zb = jnp.zeros((128,), _F32)

    def step(carry, p, last):
        xm, xo, ea_mo, ea_om = carry
        e_mo, e_om = p['mo']['edge'], p['om']['edge']
        t_mesh = _proj_tc(
            xm,
            jnp.stack([e_mo['W1'][128:256], e_om['W1'][0:128]]),
            jnp.stack([zb, e_om['b1']])[:, None, :])
        t_obj = _proj_tc(
            xo,
            jnp.stack([e_mo['W1'][0:128], e_om['W1'][128:256]]),
            jnp.stack([e_mo['b1'], zb])[:, None, :])
        tsmo, tdom = t_mesh[:nm], t_mesh[nm:]
        tdmo, tsom = t_obj[:no], t_obj[no:]
        g_mo = _gather_sc(tdmo, tsmo, smo, dmo)
        g_om = _gather_sc(tdom, tsom, som, dom)
        if last:
            eu_mo = _edge_tc(g_mo, ea_mo, e_mo, want_ea=False)
            eu_om = _edge_tc(g_om, ea_om, e_om, want_ea=False)
        else:
            eu_mo, ea_mo = _edge_tc(g_mo, ea_mo, e_mo)
            eu_om, ea_om = _edge_tc(g_om, ea_om, e_om)
        p_obj = _scatter_sc(eu_mo, dmo, no)
        p_mesh = _scatter_sc(eu_om, dom, nm)
        xo2 = _node_tc(xo, p_obj, p['mo']['node'])
        xm2 = _node_tc(xm, p_mesh, p['om']['node'])
        return (xm2, xo2, ea_mo, ea_om)

    carry = (xm, xo, edge_attr_mo.astype(_BF16), edge_attr_om.astype(_BF16))
    for si, p in enumerate(params):
        carry = step(carry, p, si == len(params) - 1)
    xm, xo = carry[0], carry[1]
    return jnp.concatenate([xm[:nm0], xo[:no0]], axis=0)
